# Initial kernel scaffold; baseline (speedup 1.0000x reference)
import jax
import jax.numpy as jnp
from jax.experimental import pallas as pl


def kernel(x, lsv_matrix):
    raise NotImplementedError("write your pallas kernel here")


# speedup vs baseline: 1.0026x; 1.0026x over previous
import jax
import jax.numpy as jnp
from jax.experimental import pallas as pl

_LSV_INDEX = 0
_SCALE = 1.0


def _add_kernel(x_ref, m_ref, o_ref):
    o_ref[...] = x_ref[...] + m_ref[_LSV_INDEX, :] * _SCALE


def kernel(x, lsv_matrix):
    b, s, d = x.shape
    rows = b * s
    x2 = x.reshape(rows, d)
    blk = 1024
    grid = (rows // blk,)
    out = pl.pallas_call(
        _add_kernel,
        grid=grid,
        in_specs=[
            pl.BlockSpec((blk, d), lambda i: (i, 0)),
            pl.BlockSpec(lsv_matrix.shape, lambda i: (0, 0)),
        ],
        out_specs=pl.BlockSpec((blk, d), lambda i: (i, 0)),
        out_shape=jax.ShapeDtypeStruct((rows, d), x.dtype),
    )(x2, lsv_matrix)
    return out.reshape(b, s, d)
